# baseline (speedup 1.0000x reference)
import functools

import jax
import jax.numpy as jnp
from jax import lax
from jax.experimental import pallas as pl
from jax.experimental.pallas import tpu as pltpu
from jax.experimental.pallas import tpu_sc as plsc

N = 10000
E = 320000
D = 128
DE = 16
B = 64
DH = D // 2

NC = 2
NS = 16
CH = 128
KB = 8

ERE = E // CH
RA = 158
RE = 79

NP = 10240
RPT = NP // NS
ZB = 64


def _sc_a_body(x0a, x0b, ei0_h, a0_out,
               zbuf, srcb, dstb, xrows, a_sh, gsems, ssems, isems):
    cid = lax.axis_index("c")
    sid = lax.axis_index("s")
    rbase = sid * RPT

    z16 = jnp.zeros((16,), jnp.float32)

    def zrow(i, _):
        zbuf[i // 4, pl.ds((i % 4) * 16, 16)] = z16
        return 0

    lax.fori_loop(0, ZB * 4, zrow, 0)

    def wait_idx(s):
        pltpu.make_async_copy(ei0_h.at[0, pl.ds(0, CH)], srcb.at[0],
                              isems.at[s]).wait()

    def wait_gather(x_h, s):
        pltpu.make_async_copy(x_h.at[pl.ds(0, CH)], xrows.at[0],
                              gsems.at[s]).wait()

    def wait_scatter(s):
        pltpu.make_async_copy(xrows.at[0], a_sh.at[pl.ds(0, CH)],
                              ssems.at[s]).wait()

    def run_a(x_h, ei_h):
        r0 = sid * RA
        nrows = jnp.clip(ERE - r0, 0, RA)
        e0 = r0 * CH

        for k in range(KB):
            pltpu.sync_copy(ei_h.at[0, pl.ds(e0 + k * CH, CH)], srcb.at[k])
            pltpu.sync_copy(ei_h.at[1, pl.ds(e0 + k * CH, CH)], dstb.at[k])
            pltpu.async_copy(x_h.at[srcb.at[k]], xrows.at[k], gsems.at[k])

        def substep(j):
            b = lax.rem(j, KB)
            b2 = lax.rem(j + KB - 2, KB)
            b3 = lax.rem(j + KB - 3, KB)
            jn2 = j + KB - 2
            jn3 = j + KB - 3

            @pl.when(j >= 2)
            def _():
                wait_scatter(b2)

                @pl.when(jn2 < nrows)
                def _():
                    pltpu.async_copy(ei_h.at[0, pl.ds(e0 + jn2 * CH, CH)],
                                     srcb.at[b2], isems.at[b2])
                    pltpu.async_copy(ei_h.at[1, pl.ds(e0 + jn2 * CH, CH)],
                                     dstb.at[b2], isems.at[b2])

            @pl.when(jnp.logical_and(j >= 3, jn3 < nrows))
            def _():
                wait_idx(b3)
                wait_idx(b3)
                pltpu.async_copy(x_h.at[srcb.at[b3]], xrows.at[b3],
                                 gsems.at[b3])

            wait_gather(x_h, b)
            pltpu.async_copy(xrows.at[b], a_sh.at[dstb.at[b]], ssems.at[b],
                             add=True)

        def body(j2, _):
            substep(2 * j2)
            substep(2 * j2 + 1)
            return 0

        lax.fori_loop(0, nrows // 2, body, 0)
        wait_scatter(lax.rem(nrows - 2, KB))
        wait_scatter(lax.rem(nrows - 1, KB))

    def phase_a(xa, xb, ei_h, a_out):
        for k in range(RPT // ZB):
            pltpu.sync_copy(zbuf, a_sh.at[pl.ds(rbase + k * ZB, ZB)])
        plsc.subcore_barrier()

        @pl.when(cid == 0)
        def _():
            run_a(xa, ei_h)

        @pl.when(cid == 1)
        def _():
            run_a(xb, ei_h)

        plsc.subcore_barrier()
        pltpu.sync_copy(a_sh.at[pl.ds(rbase, RPT)],
                        a_out.at[pl.ds(rbase, RPT), pl.ds(cid * DH, DH)])
        plsc.subcore_barrier()

    phase_a(x0a, x0b, ei0_h, a0_out)


def _sc_ae_body(ei_h, ea_h, ae_out,
                zbufe, dstb, earows, ae_sh, ssems, isems, lsems):
    cid = lax.axis_index("c")
    sid = lax.axis_index("s")
    rbase = sid * RPT

    z16 = jnp.zeros((16,), jnp.float32)

    def zerow(i, _):
        zbufe[i, :] = z16
        return 0

    lax.fori_loop(0, ZB, zerow, 0)

    def wait_idx(s):
        pltpu.make_async_copy(ei_h.at[0, pl.ds(0, CH)], dstb.at[0],
                              isems.at[s]).wait()

    def wait_eload(s):
        pltpu.make_async_copy(ea_h.at[pl.ds(0, CH)], earows.at[0],
                              lsems.at[s]).wait()

    def wait_escatter(s):
        pltpu.make_async_copy(earows.at[0], ae_sh.at[pl.ds(0, CH)],
                              ssems.at[s]).wait()

    def run_ae(ei_h, ea_h):
        r0 = cid * (NS * RE) + sid * RE
        nrows = jnp.clip(ERE - r0, 0, RE)
        e0 = r0 * CH

        for k in range(KB):
            pltpu.sync_copy(ei_h.at[1, pl.ds(e0 + k * CH, CH)], dstb.at[k])
            pltpu.async_copy(ea_h.at[pl.ds(e0 + k * CH, CH)], earows.at[k],
                             lsems.at[k])

        def body(j, _):
            b = lax.rem(j, KB)
            b2 = lax.rem(j + KB - 2, KB)
            jn2 = j + KB - 2

            @pl.when(j >= 2)
            def _():
                wait_escatter(b2)

                @pl.when(jn2 < nrows)
                def _():
                    pltpu.async_copy(ei_h.at[1, pl.ds(e0 + jn2 * CH, CH)],
                                     dstb.at[b2], isems.at[b2])
                    pltpu.async_copy(ea_h.at[pl.ds(e0 + jn2 * CH, CH)],
                                     earows.at[b2], lsems.at[b2])

            wait_eload(b)

            @pl.when(j >= KB)
            def _():
                wait_idx(b)

            pltpu.async_copy(earows.at[b], ae_sh.at[dstb.at[b]], ssems.at[b],
                             add=True)
            return 0

        lax.fori_loop(0, nrows, body, 0)
        wait_escatter(lax.rem(nrows - 2, KB))
        wait_escatter(lax.rem(nrows - 1, KB))

    for k in range(RPT // ZB):
        pltpu.sync_copy(zbufe, ae_sh.at[pl.ds(rbase + k * ZB, ZB)])
    plsc.subcore_barrier()
    run_ae(ei_h, ea_h)
    plsc.subcore_barrier()
    pltpu.sync_copy(ae_sh.at[pl.ds(rbase, RPT)],
                    ae_out.at[pl.ds(rbase, RPT), pl.ds(cid * DE, DE)])


_SC_MESH = dict(core_axis_name="c", subcore_axis_name="s",
                num_cores=NC, num_subcores=NS)


@functools.cache
def _make_sc_a():
    return pl.kernel(
        _sc_a_body,
        out_type=jax.ShapeDtypeStruct((NP, D), jnp.float32),
        mesh=plsc.VectorSubcoreMesh(**_SC_MESH),
        compiler_params=pltpu.CompilerParams(use_tc_tiling_on_sc=False),
        scratch_types=[
            pltpu.VMEM((ZB, DH), jnp.float32),
            pltpu.VMEM((KB, CH), jnp.int32),
            pltpu.VMEM((KB, CH), jnp.int32),
            pltpu.VMEM((KB, CH, DH), jnp.float32),
            pltpu.VMEM_SHARED((NP, DH), jnp.float32),
            pltpu.SemaphoreType.DMA((KB,)),
            pltpu.SemaphoreType.DMA((KB,)),
            pltpu.SemaphoreType.DMA((KB,)),
        ],
    )


@functools.cache
def _make_sc_ae():
    return pl.kernel(
        _sc_ae_body,
        out_type=jax.ShapeDtypeStruct((NP, D), jnp.float32),
        mesh=plsc.VectorSubcoreMesh(**_SC_MESH),
        compiler_params=pltpu.CompilerParams(use_tc_tiling_on_sc=False),
        scratch_types=[
            pltpu.VMEM((ZB, DE), jnp.float32),
            pltpu.VMEM((KB, CH), jnp.int32),
            pltpu.VMEM((KB, CH, DE), jnp.float32),
            pltpu.VMEM_SHARED((NP, DE), jnp.float32),
            pltpu.SemaphoreType.DMA((KB,)),
            pltpu.SemaphoreType.DMA((KB,)),
            pltpu.SemaphoreType.DMA((KB,)),
        ],
    )


def _head_body(x_ref, a_ref, ae_ref, batch_ref, lab_ref,
               ws_ref, wn_ref, we_ref, out_ref):
    rep = jnp.dot(x_ref[...], ws_ref[...], preferred_element_type=jnp.float32)
    rep += jnp.dot(a_ref[:N, :], wn_ref[...],
                   preferred_element_type=jnp.float32)
    rep += jnp.dot(ae_ref[:N, :DE] + ae_ref[:N, DE:2 * DE], we_ref[...],
                   preferred_element_type=jnp.float32)
    rep = jnp.maximum(rep, 0.0)

    batch = batch_ref[...].reshape(1, N)
    lab = lab_ref[...].reshape(1, N)
    iota = lax.broadcasted_iota(jnp.int32, (B, 1), 0)
    m = (batch == iota).astype(jnp.float32)
    cond = (lab == -1)
    mc = m * cond.astype(jnp.float32)
    bnext = jnp.concatenate([batch[:, 1:], jnp.full((1, 1), B, jnp.int32)],
                            axis=1)
    lastc = jnp.where((bnext != batch) & cond, 1.0, 0.0)
    flag = lax.dot_general(m, lastc, (((1,), (1,)), ((), ())),
                           preferred_element_type=jnp.float32)
    cnt = jnp.sum(mc, axis=1, keepdims=True)
    csum = jnp.dot(mc, rep, preferred_element_type=jnp.float32)
    pool = jnp.where(flag > 0, csum / jnp.maximum(cnt, 1.0), 0.0)
    out2 = lax.dot_general(m, pool, (((0,), (0,)), ((), ())),
                           preferred_element_type=jnp.float32)
    out_ref[:, :D] = rep
    out_ref[:, D:] = out2


_head_call = pl.pallas_call(
    _head_body,
    out_shape=jax.ShapeDtypeStruct((N, 2 * D), jnp.float32),
    compiler_params=pltpu.CompilerParams(vmem_limit_bytes=128 * 1024 * 1024),
)


def kernel(x0, edge_index0, edge_attr0, batch0, primary_label0, mlabes0,
           x1, edge_index1, edge_attr1, batch1, primary_label1,
           W_self, W_nbr, W_edge):
    a0 = _make_sc_a()(x0[:, :DH], x0[:, DH:], edge_index0)
    ae0 = _make_sc_ae()(edge_index0, edge_attr0)
    ea1d, _ = lax.optimization_barrier((edge_attr1, a0))
    a1 = _make_sc_a()(x1[:, :DH], x1[:, DH:], edge_index1)
    ae1 = _make_sc_ae()(edge_index1, ea1d)
    mask_head = _head_call(x0, a0, ae0, batch0, primary_label0,
                           W_self, W_nbr, W_edge)
    indenti_head = _head_call(x1, a1, ae1, batch1, primary_label1,
                              W_self, W_nbr, W_edge)
    return (mask_head, mlabes0, indenti_head, primary_label1)


# speedup vs baseline: 1.1410x; 1.1410x over previous
import functools

import jax
import jax.numpy as jnp
from jax import lax
from jax.experimental import pallas as pl
from jax.experimental.pallas import tpu as pltpu
from jax.experimental.pallas import tpu_sc as plsc

N = 10000
E = 320000
D = 128
DE = 16
B = 64
DH = D // 2

NC = 2
NS = 16
CH = 128
KB = 8

ERE = E // CH
RA = 158
RE = 79

NP = 10240
RPT = NP // NS
ZB = 64


def _sc_a_body(x0a, x0b, ei0_h, a0_out,
               zbuf, srcb, dstb, xrows, a_sh, gsems, ssems, isems):
    cid = lax.axis_index("c")
    sid = lax.axis_index("s")
    rbase = sid * RPT

    z16 = jnp.zeros((16,), jnp.float32)

    def zrow(i, _):
        zbuf[i // 4, pl.ds((i % 4) * 16, 16)] = z16
        return 0

    lax.fori_loop(0, ZB * 4, zrow, 0)

    def wait_idx(s):
        pltpu.make_async_copy(ei0_h.at[0, pl.ds(0, CH)], srcb.at[0],
                              isems.at[s]).wait()

    def wait_gather(x_h, s):
        pltpu.make_async_copy(x_h.at[pl.ds(0, CH)], xrows.at[0],
                              gsems.at[s]).wait()

    def wait_scatter(s):
        pltpu.make_async_copy(xrows.at[0], a_sh.at[pl.ds(0, CH)],
                              ssems.at[s]).wait()

    def run_a(x_h, ei_h):
        r0 = sid * RA
        nrows = jnp.clip(ERE - r0, 0, RA)
        e0 = r0 * CH

        for k in range(KB):
            pltpu.sync_copy(ei_h.at[0, pl.ds(e0 + k * CH, CH)], srcb.at[k])
            pltpu.sync_copy(ei_h.at[1, pl.ds(e0 + k * CH, CH)], dstb.at[k])
            pltpu.async_copy(x_h.at[srcb.at[k]], xrows.at[k], gsems.at[k])

        def substep(j):
            b = lax.rem(j, KB)
            b2 = lax.rem(j + KB - 2, KB)
            b3 = lax.rem(j + KB - 3, KB)
            jn2 = j + KB - 2
            jn3 = j + KB - 3

            @pl.when(j >= 2)
            def _():
                wait_scatter(b2)

                @pl.when(jn2 < nrows)
                def _():
                    pltpu.async_copy(ei_h.at[0, pl.ds(e0 + jn2 * CH, CH)],
                                     srcb.at[b2], isems.at[b2])
                    pltpu.async_copy(ei_h.at[1, pl.ds(e0 + jn2 * CH, CH)],
                                     dstb.at[b2], isems.at[b2])

            @pl.when(jnp.logical_and(j >= 3, jn3 < nrows))
            def _():
                wait_idx(b3)
                wait_idx(b3)
                pltpu.async_copy(x_h.at[srcb.at[b3]], xrows.at[b3],
                                 gsems.at[b3])

            wait_gather(x_h, b)
            pltpu.async_copy(xrows.at[b], a_sh.at[dstb.at[b]], ssems.at[b],
                             add=True)

        def body(j2, _):
            substep(2 * j2)
            substep(2 * j2 + 1)
            return 0

        lax.fori_loop(0, nrows // 2, body, 0)
        wait_scatter(lax.rem(nrows - 2, KB))
        wait_scatter(lax.rem(nrows - 1, KB))

    def phase_a(xa, xb, ei_h, a_out):
        for k in range(RPT // ZB):
            pltpu.sync_copy(zbuf, a_sh.at[pl.ds(rbase + k * ZB, ZB)])
        plsc.subcore_barrier()

        @pl.when(cid == 0)
        def _():
            run_a(xa, ei_h)

        @pl.when(cid == 1)
        def _():
            run_a(xb, ei_h)

        plsc.subcore_barrier()
        pltpu.sync_copy(a_sh.at[pl.ds(rbase, RPT)],
                        a_out.at[pl.ds(rbase, RPT), pl.ds(cid * DH, DH)])
        plsc.subcore_barrier()

    phase_a(x0a, x0b, ei0_h, a0_out)


def _sc_ae_body(ei_h, ea_h, ae_out,
                zbufe, dstb, earows, ae_sh, ssems, isems, lsems):
    cid = lax.axis_index("c")
    sid = lax.axis_index("s")
    rbase = sid * RPT

    z16 = jnp.zeros((16,), jnp.float32)

    def zerow(i, _):
        zbufe[i, :] = z16
        return 0

    lax.fori_loop(0, ZB, zerow, 0)

    def wait_idx(s):
        pltpu.make_async_copy(ei_h.at[0, pl.ds(0, CH)], dstb.at[0],
                              isems.at[s]).wait()

    def wait_eload(s):
        pltpu.make_async_copy(ea_h.at[pl.ds(0, CH)], earows.at[0],
                              lsems.at[s]).wait()

    def wait_escatter(s):
        pltpu.make_async_copy(earows.at[0], ae_sh.at[pl.ds(0, CH)],
                              ssems.at[s]).wait()

    def run_ae(ei_h, ea_h):
        r0 = cid * (NS * RE) + sid * RE
        nrows = jnp.clip(ERE - r0, 0, RE)
        e0 = r0 * CH

        for k in range(KB):
            pltpu.sync_copy(ei_h.at[1, pl.ds(e0 + k * CH, CH)], dstb.at[k])
            pltpu.async_copy(ea_h.at[pl.ds(e0 + k * CH, CH)], earows.at[k],
                             lsems.at[k])

        def body(j, _):
            b = lax.rem(j, KB)
            b2 = lax.rem(j + KB - 2, KB)
            jn2 = j + KB - 2

            @pl.when(j >= 2)
            def _():
                wait_escatter(b2)

                @pl.when(jn2 < nrows)
                def _():
                    pltpu.async_copy(ei_h.at[1, pl.ds(e0 + jn2 * CH, CH)],
                                     dstb.at[b2], isems.at[b2])
                    pltpu.async_copy(ea_h.at[pl.ds(e0 + jn2 * CH, CH)],
                                     earows.at[b2], lsems.at[b2])

            wait_eload(b)

            @pl.when(j >= KB)
            def _():
                wait_idx(b)

            pltpu.async_copy(earows.at[b], ae_sh.at[dstb.at[b]], ssems.at[b],
                             add=True)
            return 0

        lax.fori_loop(0, nrows, body, 0)
        wait_escatter(lax.rem(nrows - 2, KB))
        wait_escatter(lax.rem(nrows - 1, KB))

    for k in range(RPT // ZB):
        pltpu.sync_copy(zbufe, ae_sh.at[pl.ds(rbase + k * ZB, ZB)])
    plsc.subcore_barrier()
    run_ae(ei_h, ea_h)
    plsc.subcore_barrier()
    pltpu.sync_copy(ae_sh.at[pl.ds(rbase, RPT)],
                    ae_out.at[pl.ds(rbase, RPT), pl.ds(cid * DE, DE)])


_SC_MESH = dict(core_axis_name="c", subcore_axis_name="s",
                num_cores=NC, num_subcores=NS)


@functools.cache
def _make_sc_a():
    return pl.kernel(
        _sc_a_body,
        out_type=jax.ShapeDtypeStruct((NP, D), jnp.float32),
        mesh=plsc.VectorSubcoreMesh(**_SC_MESH),
        compiler_params=pltpu.CompilerParams(use_tc_tiling_on_sc=False),
        scratch_types=[
            pltpu.VMEM((ZB, DH), jnp.float32),
            pltpu.VMEM((KB, CH), jnp.int32),
            pltpu.VMEM((KB, CH), jnp.int32),
            pltpu.VMEM((KB, CH, DH), jnp.float32),
            pltpu.VMEM_SHARED((NP, DH), jnp.float32),
            pltpu.SemaphoreType.DMA((KB,)),
            pltpu.SemaphoreType.DMA((KB,)),
            pltpu.SemaphoreType.DMA((KB,)),
        ],
    )


@functools.cache
def _make_sc_ae():
    return pl.kernel(
        _sc_ae_body,
        out_type=jax.ShapeDtypeStruct((NP, D), jnp.float32),
        mesh=plsc.VectorSubcoreMesh(**_SC_MESH),
        compiler_params=pltpu.CompilerParams(use_tc_tiling_on_sc=False),
        scratch_types=[
            pltpu.VMEM((ZB, DE), jnp.float32),
            pltpu.VMEM((KB, CH), jnp.int32),
            pltpu.VMEM((KB, CH, DE), jnp.float32),
            pltpu.VMEM_SHARED((NP, DE), jnp.float32),
            pltpu.SemaphoreType.DMA((KB,)),
            pltpu.SemaphoreType.DMA((KB,)),
            pltpu.SemaphoreType.DMA((KB,)),
        ],
    )


def _head_body(x_ref, a_ref, ae_ref, batch_ref, lab_ref,
               ws_ref, wn_ref, we_ref, out_ref):
    rep = jnp.dot(x_ref[...], ws_ref[...], preferred_element_type=jnp.float32)
    rep += jnp.dot(a_ref[:N, :], wn_ref[...],
                   preferred_element_type=jnp.float32)
    rep += jnp.dot(ae_ref[:N, :DE] + ae_ref[:N, DE:2 * DE], we_ref[...],
                   preferred_element_type=jnp.float32)
    rep = jnp.maximum(rep, 0.0)

    batch = batch_ref[...].reshape(1, N)
    lab = lab_ref[...].reshape(1, N)
    iota = lax.broadcasted_iota(jnp.int32, (B, 1), 0)
    m = (batch == iota).astype(jnp.float32)
    cond = (lab == -1)
    mc = m * cond.astype(jnp.float32)
    bnext = jnp.concatenate([batch[:, 1:], jnp.full((1, 1), B, jnp.int32)],
                            axis=1)
    lastc = jnp.where((bnext != batch) & cond, 1.0, 0.0)
    flag = lax.dot_general(m, lastc, (((1,), (1,)), ((), ())),
                           preferred_element_type=jnp.float32)
    cnt = jnp.sum(mc, axis=1, keepdims=True)
    csum = jnp.dot(mc, rep, preferred_element_type=jnp.float32)
    pool = jnp.where(flag > 0, csum / jnp.maximum(cnt, 1.0), 0.0)
    out2 = lax.dot_general(m, pool, (((0,), (0,)), ((), ())),
                           preferred_element_type=jnp.float32)
    out_ref[:, :D] = rep
    out_ref[:, D:] = out2


_head_call = pl.pallas_call(
    _head_body,
    out_shape=jax.ShapeDtypeStruct((N, 2 * D), jnp.float32),
    compiler_params=pltpu.CompilerParams(vmem_limit_bytes=128 * 1024 * 1024),
)


def kernel(x0, edge_index0, edge_attr0, batch0, primary_label0, mlabes0,
           x1, edge_index1, edge_attr1, batch1, primary_label1,
           W_self, W_nbr, W_edge):
    a0 = _make_sc_a()(x0[:, :DH], x0[:, DH:], edge_index0)
    ae0 = _make_sc_ae()(edge_index0, edge_attr0)
    a1 = _make_sc_a()(x1[:, :DH], x1[:, DH:], edge_index1)
    ae1 = _make_sc_ae()(edge_index1, edge_attr1)
    mask_head = _head_call(x0, a0, ae0, batch0, primary_label0,
                           W_self, W_nbr, W_edge)
    indenti_head = _head_call(x1, a1, ae1, batch1, primary_label1,
                              W_self, W_nbr, W_edge)
    return (mask_head, mlabes0, indenti_head, primary_label1)
